# baseline (speedup 1.0000x reference)
import jax
import jax.numpy as jnp
from jax.experimental import pallas as pl
from jax.experimental.pallas import tpu as pltpu

_CH = 8
_NBUF = 8


def _posadd_kernel(pos_ref, x_hbm, o_hbm, in_buf, out_buf, in_sem, out_sem):
    n = x_hbm.shape[0] // _CH

    def in_copy(i, slot):
        return pltpu.make_async_copy(
            x_hbm.at[pl.ds(i * _CH, _CH)], in_buf.at[slot], in_sem.at[slot]
        )

    def out_copy(i, slot):
        return pltpu.make_async_copy(
            out_buf.at[slot], o_hbm.at[pl.ds(i * _CH, _CH)], out_sem.at[slot]
        )

    for k in range(_NBUF):
        in_copy(k, k).start()

    def body(i, carry):
        slot = jax.lax.rem(i, _NBUF)
        in_copy(i, slot).wait()

        @pl.when(i >= _NBUF)
        def _():
            out_copy(i - _NBUF, slot).wait()

        out_buf[slot] = in_buf[slot] + pos_ref[0:500, :][None]
        out_copy(i, slot).start()

        @pl.when(i + _NBUF < n)
        def _():
            in_copy(i + _NBUF, slot).start()

        return carry

    jax.lax.fori_loop(0, n, body, 0)

    for k in range(max(0, n - _NBUF), n):
        out_copy(k, k % _NBUF).wait()


def kernel(x, pos_table):
    B, S, D = x.shape
    return pl.pallas_call(
        _posadd_kernel,
        in_specs=[
            pl.BlockSpec((pos_table.shape[0], D), lambda: (0, 0)),
            pl.BlockSpec(memory_space=pl.ANY),
        ],
        out_specs=pl.BlockSpec(memory_space=pl.ANY),
        out_shape=jax.ShapeDtypeStruct((B, S, D), x.dtype),
        scratch_shapes=[
            pltpu.VMEM((_NBUF, _CH, S, D), jnp.float32),
            pltpu.VMEM((_NBUF, _CH, S, D), jnp.float32),
            pltpu.SemaphoreType.DMA((_NBUF,)),
            pltpu.SemaphoreType.DMA((_NBUF,)),
        ],
    )(pos_table, x)


# speedup vs baseline: 3.3044x; 3.3044x over previous
import jax
import jax.numpy as jnp
from jax.experimental import pallas as pl

_SB = 8


def _posadd_kernel(x_ref, pos_ref, o_ref):
    o_ref[...] = x_ref[...] + pos_ref[...][:, None, :]


def kernel(x, pos_table):
    B, S, D = x.shape
    xt = jnp.transpose(x, (1, 0, 2))
    out_t = pl.pallas_call(
        _posadd_kernel,
        grid=(pl.cdiv(S, _SB),),
        in_specs=[
            pl.BlockSpec((_SB, B, D), lambda i: (i, 0, 0)),
            pl.BlockSpec((_SB, D), lambda i: (i, 0)),
        ],
        out_specs=pl.BlockSpec((_SB, B, D), lambda i: (i, 0, 0)),
        out_shape=jax.ShapeDtypeStruct((S, B, D), x.dtype),
    )(xt, pos_table)
    return jnp.transpose(out_t, (1, 0, 2))


# speedup vs baseline: 3.3054x; 1.0003x over previous
import jax
import jax.numpy as jnp
from jax.experimental import pallas as pl

_SB = 8


def _posadd_kernel(x_ref, pos_ref, o_ref):
    i = pl.program_id(0)
    pos = pos_ref[pl.ds(i * _SB, _SB), :]
    o_ref[...] = x_ref[...] + pos[:, None, :]


def kernel(x, pos_table):
    B, S, D = x.shape
    xt = jnp.transpose(x, (1, 0, 2))
    out_t = pl.pallas_call(
        _posadd_kernel,
        grid=(pl.cdiv(S, _SB),),
        in_specs=[
            pl.BlockSpec((_SB, B, D), lambda i: (i, 0, 0)),
            pl.BlockSpec((512, D), lambda i: (0, 0)),
        ],
        out_specs=pl.BlockSpec((_SB, B, D), lambda i: (i, 0, 0)),
        out_shape=jax.ShapeDtypeStruct((S, B, D), x.dtype),
    )(xt, pos_table)
    return jnp.transpose(out_t, (1, 0, 2))
